# baseline (speedup 1.0000x reference)
import functools

import jax
import jax.numpy as jnp
from jax import lax
from jax.experimental import pallas as pl
from jax.experimental.pallas import tpu as pltpu
from jax.experimental.pallas import tpu_sc as plsc

S = 2048
D = 768
H = 12
DH = 64
E = 8
CAP = 307
CAPP = 320
RB = 256
QB = 512
NW = 32
TPW = S // NW

F32 = jnp.float32


def _dot(a, b, dims):
    return lax.dot_general(a.astype(jnp.bfloat16), b.astype(jnp.bfloat16),
                           (dims, ((), ())),
                           preferred_element_type=F32)


def _dot_hi(a, b, dims):
    return lax.dot_general(a, b, (dims, ((), ())),
                           precision=lax.Precision.HIGHEST,
                           preferred_element_type=F32)


def _ln(x, g, b):
    mu = jnp.mean(x, axis=-1, keepdims=True)
    var = jnp.mean((x - mu) ** 2, axis=-1, keepdims=True)
    return (x - mu) / jnp.sqrt(var + 1e-5) * g + b


@functools.lru_cache(maxsize=None)
def _make_emb_gather():
    mesh = plsc.VectorSubcoreMesh(
        core_axis_name="c", subcore_axis_name="s",
        num_cores=2, num_subcores=16)

    @functools.partial(
        pl.kernel,
        out_type=jax.ShapeDtypeStruct((S, D), F32),
        mesh=mesh,
        scratch_types=[
            pltpu.VMEM((TPW,), jnp.int32),
            pltpu.VMEM((TPW, D), F32),
            pltpu.SemaphoreType.DMA,
        ],
    )
    def _emb_gather(table_hbm, idx_hbm, out_hbm, idx_v, rows_v, sem):
        wid = lax.axis_index("s") * 2 + lax.axis_index("c")
        base = wid * TPW
        pltpu.sync_copy(idx_hbm.at[pl.ds(base, TPW)], idx_v)
        pltpu.async_copy(table_hbm.at[idx_v], rows_v, sem).wait()
        pltpu.sync_copy(rows_v, out_hbm.at[pl.ds(base, TPW)])

    return _emb_gather


def _qkv_body(h_ref, g_ref, b_ref, w_ref, wb_ref, q_ref, k_ref, v_ref):
    hn = _ln(h_ref[...], g_ref[...], b_ref[...])
    qkv = _dot_hi(hn, w_ref[...], ((1,), (1,))) + wb_ref[...]
    q_ref[...] = qkv[:, :D]
    k_ref[...] = qkv[:, D:2 * D]
    v_ref[...] = qkv[:, 2 * D:]


_qkv = pl.pallas_call(
    _qkv_body,
    grid=(S // RB,),
    in_specs=[
        pl.BlockSpec((RB, D), lambda i: (i, 0)),
        pl.BlockSpec((1, D), lambda i: (0, 0)),
        pl.BlockSpec((1, D), lambda i: (0, 0)),
        pl.BlockSpec((3 * D, D), lambda i: (0, 0)),
        pl.BlockSpec((1, 3 * D), lambda i: (0, 0)),
    ],
    out_specs=[pl.BlockSpec((RB, D), lambda i: (i, 0))] * 3,
    out_shape=[jax.ShapeDtypeStruct((S, D), F32)] * 3,
)


def _attn_body(q_ref, k_ref, v_ref, ow_ref, ob_ref, res_ref, o_ref):
    ohs = []
    for hh in range(H):
        s = _dot_hi(q_ref[hh], k_ref[hh], ((1,), (1,))) * 0.125
        m = jnp.max(s, axis=-1, keepdims=True)
        ex = jnp.exp(s - m)
        p = ex / jnp.sum(ex, axis=-1, keepdims=True)
        ohs.append(_dot_hi(p, v_ref[hh], ((1,), (0,))))
    o_cat = jnp.concatenate(ohs, axis=-1)
    o_ref[...] = (_dot_hi(o_cat, ow_ref[...], ((1,), (1,))) + ob_ref[...]
                  + res_ref[...])


_attn = pl.pallas_call(
    _attn_body,
    grid=(S // QB,),
    in_specs=[
        pl.BlockSpec((H, QB, DH), lambda i: (0, i, 0)),
        pl.BlockSpec((H, S, DH), lambda i: (0, 0, 0)),
        pl.BlockSpec((H, S, DH), lambda i: (0, 0, 0)),
        pl.BlockSpec((D, D), lambda i: (0, 0)),
        pl.BlockSpec((1, D), lambda i: (0, 0)),
        pl.BlockSpec((QB, D), lambda i: (i, 0)),
    ],
    out_specs=pl.BlockSpec((QB, D), lambda i: (i, 0)),
    out_shape=jax.ShapeDtypeStruct((S, D), F32),
)


def _route_body(h_ref, g_ref, b_ref, rw_ref,
                hn_ref, sc_ref, slot_ref, rp_ref, aux_ref, cnt, pacc):
    i = pl.program_id(0)

    @pl.when(i == 0)
    def _():
        cnt[...] = jnp.zeros((1, E), F32)
        pacc[...] = jnp.zeros((1, E), F32)

    hn = _ln(h_ref[...], g_ref[...], b_ref[...])
    hn_ref[...] = hn
    logits = _dot_hi(hn, rw_ref[...], ((1,), (0,)))
    m = jnp.max(logits, axis=-1, keepdims=True)
    ex = jnp.exp(logits - m)
    probs = ex / jnp.sum(ex, axis=-1, keepdims=True)
    top = jnp.max(probs, axis=-1, keepdims=True)
    io = lax.broadcasted_iota(jnp.int32, (RB, E), 1)
    fi = jnp.min(jnp.where(probs >= top, io, E), axis=-1, keepdims=True)
    onehot = (io == fi).astype(F32)
    r = lax.broadcasted_iota(jnp.int32, (RB, RB), 0)
    c = lax.broadcasted_iota(jnp.int32, (RB, RB), 1)
    tri = (c <= r).astype(F32)
    pos = _dot(tri, onehot, ((1,), (0,))) + cnt[...]
    keep = jnp.where((pos <= CAP) & (onehot > 0), 1.0, 0.0).astype(F32)
    active = jnp.sum(keep, axis=-1, keepdims=True)
    rp = top * active
    sc_ref[...] = keep * rp
    rp_ref[...] = rp
    slotf = jnp.sum(keep * (io.astype(F32) * CAPP + pos - 1.0),
                    axis=-1, keepdims=True)
    slot_ref[...] = jnp.where(active > 0, slotf,
                              float(E * CAPP - 1)).astype(jnp.int32)
    cnt[...] = cnt[...] + jnp.sum(onehot, axis=0, keepdims=True)
    pacc[...] = pacc[...] + jnp.sum(probs, axis=0, keepdims=True)

    @pl.when(i == pl.num_programs(0) - 1)
    def _():
        aux_ref[...] = (E * jnp.sum(cnt[...] * pacc[...]) /
                        (S * S)).reshape(1, 1)


_route = pl.pallas_call(
    _route_body,
    grid=(S // RB,),
    in_specs=[
        pl.BlockSpec((RB, D), lambda i: (i, 0)),
        pl.BlockSpec((1, D), lambda i: (0, 0)),
        pl.BlockSpec((1, D), lambda i: (0, 0)),
        pl.BlockSpec((D, E), lambda i: (0, 0)),
    ],
    out_specs=[
        pl.BlockSpec((RB, D), lambda i: (i, 0)),
        pl.BlockSpec((RB, E), lambda i: (i, 0)),
        pl.BlockSpec((RB, 1), lambda i: (i, 0)),
        pl.BlockSpec((RB, 1), lambda i: (i, 0)),
        pl.BlockSpec((1, 1), lambda i: (0, 0)),
    ],
    out_shape=[
        jax.ShapeDtypeStruct((S, D), F32),
        jax.ShapeDtypeStruct((S, E), F32),
        jax.ShapeDtypeStruct((S, 1), jnp.int32),
        jax.ShapeDtypeStruct((S, 1), F32),
        jax.ShapeDtypeStruct((1, 1), F32),
    ],
    scratch_shapes=[pltpu.VMEM((1, E), F32), pltpu.VMEM((1, E), F32)],
)


def _moe_dense_body(hn_ref, ew_ref, eb_ref, sc_ref, res_ref, o_ref):
    e = pl.program_id(1)
    y = _dot_hi(hn_ref[...], ew_ref[0], ((1,), (0,))) + eb_ref[0]
    sel = (lax.broadcasted_iota(jnp.int32, (1, E), 1) == e).astype(F32)
    se = jnp.sum(sc_ref[...] * sel, axis=-1, keepdims=True)

    @pl.when(e == 0)
    def _():
        o_ref[...] = se * y + res_ref[...]

    @pl.when(e > 0)
    def _():
        o_ref[...] = o_ref[...] + se * y


_moe_dense = pl.pallas_call(
    _moe_dense_body,
    grid=(S // RB, E),
    in_specs=[
        pl.BlockSpec((RB, D), lambda r, e: (r, 0)),
        pl.BlockSpec((1, D, D), lambda r, e: (e, 0, 0)),
        pl.BlockSpec((1, 1, D), lambda r, e: (e, 0, 0)),
        pl.BlockSpec((RB, E), lambda r, e: (r, 0)),
        pl.BlockSpec((RB, D), lambda r, e: (r, 0)),
    ],
    out_specs=pl.BlockSpec((RB, D), lambda r, e: (r, 0)),
    out_shape=jax.ShapeDtypeStruct((S, D), F32),
)


def kernel(x, emb, in_proj_w, in_proj_b, out_proj_w, out_proj_b,
           attn_gamma, attn_beta, moe_gamma, moe_beta,
           expert_w, expert_b, router_w):
    idx = x.reshape(S).astype(jnp.int32)
    h = _make_emb_gather()(emb, idx)
    ipb = in_proj_b.reshape(1, 3 * D)
    opb = out_proj_b.reshape(1, D)
    eb3 = expert_b.reshape(E, 1, D)
    aux_total = jnp.float32(0.0)
    for i in range(2):
        q, k, v = _qkv(h, attn_gamma[i].reshape(1, D),
                       attn_beta[i].reshape(1, D), in_proj_w, ipb)
        q3 = q.reshape(S, H, DH).transpose(1, 0, 2)
        k3 = k.reshape(S, H, DH).transpose(1, 0, 2)
        v3 = v.reshape(S, H, DH).transpose(1, 0, 2)
        h = _attn(q3, k3, v3, out_proj_w, opb, h)
        hn_m, scale, slot, rp, aux = _route(
            h, moe_gamma[i].reshape(1, D), moe_beta[i].reshape(1, D),
            router_w)
        h = _moe_dense(hn_m, expert_w, eb3, scale, h)
        aux_total = aux_total + aux[0, 0]
    return h.reshape(1, S, D), aux_total
